# baseline (speedup 1.0000x reference)
import jax
import jax.numpy as jnp
from jax import lax
from jax.experimental import pallas as pl
from jax.experimental.pallas import tpu as pltpu
from jax.experimental.pallas import tpu_sc as plsc

N_CORES = 2
N_SUBCORES = 16
N_WORKERS = N_CORES * N_SUBCORES

BATCH = 1024
SEQ = 200
D_MODEL = 128
HALF = SEQ // 2
CHUNKS = BATCH * 2 // N_WORKERS
NSLOT = 4
LANES = 16


def _emb_body(ids_hbm, w_hbm, pos_hbm, out_hbm,
              idx_all, pos_v, buf,
              g0, g1, g2, g3, s0, s1, s2, s3):
    gsem = (g0, g1, g2, g3)
    ssem = (s0, s1, s2, s3)
    c = lax.axis_index("c")
    s = lax.axis_index("s")
    wid = s * N_CORES + c
    base = wid * CHUNKS

    pltpu.sync_copy(ids_hbm.at[pl.ds(base, CHUNKS)], idx_all)
    pltpu.sync_copy(pos_hbm, pos_v)

    def fire_gather(t, slot):
        pltpu.async_copy(w_hbm.at[idx_all.at[t]], buf.at[slot], gsem[slot])

    def wait_gather(t, slot):
        pltpu.make_async_copy(
            w_hbm.at[idx_all.at[t]], buf.at[slot], gsem[slot]).wait()

    def fire_store(t, slot):
        pltpu.async_copy(buf.at[slot], out_hbm.at[base + t], ssem[slot])

    def wait_store(t, slot):
        pltpu.make_async_copy(
            buf.at[slot], out_hbm.at[base + t], ssem[slot]).wait()

    def add_pos(slot, h):
        def add_rows(g, carry):
            for u in range(2):
                r = g * 2 + u
                for cc in range(D_MODEL // LANES):
                    sl = pl.ds(cc * LANES, LANES)
                    plsc.addupdate(buf.at[slot, r, sl], pos_v[h, r, sl])
            return carry
        lax.fori_loop(0, HALF // 2, add_rows, 0)

    for b in range(NSLOT - 1):
        fire_gather(b, b)

    def outer(g, carry):
        for b in range(NSLOT):
            t = g * NSLOT + b
            wait_gather(t, b)
            add_pos(b, t & 1)
            fire_store(t, b)
            nxt = t + NSLOT - 1
            ns = (b + NSLOT - 1) % NSLOT

            @pl.when(nxt < CHUNKS)
            def _fire():
                @pl.when(t >= 1)
                def _drain():
                    wait_store(t - 1, ns)
                fire_gather(nxt, ns)
        return carry

    lax.fori_loop(0, CHUNKS // NSLOT, outer, 0)

    for b in range(NSLOT):
        wait_store(CHUNKS - NSLOT + b, b)


@jax.jit
def kernel(input_ids, W, pos_table):
    ids = input_ids.reshape(BATCH * 2, HALF)
    pos = pos_table[:SEQ].reshape(2, HALF, D_MODEL)
    run = pl.kernel(
        _emb_body,
        mesh=plsc.VectorSubcoreMesh(core_axis_name="c", subcore_axis_name="s"),
        out_type=jax.ShapeDtypeStruct((BATCH * 2, HALF, D_MODEL), jnp.float32),
        scratch_types=[
            pltpu.VMEM((CHUNKS, HALF), jnp.int32),
            pltpu.VMEM((2, HALF, D_MODEL), jnp.float32),
            pltpu.VMEM((NSLOT, HALF, D_MODEL), jnp.float32),
        ] + [pltpu.SemaphoreType.DMA] * (2 * NSLOT),
    )
    out = run(ids, W, pos)
    return out.reshape(BATCH, SEQ, D_MODEL)


# speedup vs baseline: 1.9189x; 1.9189x over previous
import jax
import jax.numpy as jnp
from jax import lax
from jax.experimental import pallas as pl
from jax.experimental.pallas import tpu as pltpu
from jax.experimental.pallas import tpu_sc as plsc

N_CORES = 2
N_SUBCORES = 16
N_WORKERS = N_CORES * N_SUBCORES

BATCH = 1024
SEQ = 200
D_MODEL = 128
SEQ_PER_W = BATCH // N_WORKERS
CHUNKS = 2 * SEQ_PER_W
CLEN = 104
STRIDE = 96
NSLOT = 4
LANES = 16


def _emb_body(ids_hbm, w_hbm, pos_hbm, out_hbm,
              idx_all, pos_v, buf,
              g0, g1, g2, g3, s0, s1, s2, s3):
    gsem = (g0, g1, g2, g3)
    ssem = (s0, s1, s2, s3)
    c = lax.axis_index("c")
    s = lax.axis_index("s")
    wid = s * N_CORES + c
    base = wid * CHUNKS

    pltpu.sync_copy(ids_hbm.at[pl.ds(base, CHUNKS)], idx_all)
    pltpu.sync_copy(pos_hbm, pos_v)

    def _gather_args(t, slot):
        return w_hbm.at[idx_all.at[t]], buf.at[slot], gsem[slot]

    def _store_args(t, slot, h):
        n = CLEN if h else STRIDE
        seq = wid * SEQ_PER_W + (t >> 1)
        dst = out_hbm.at[seq, pl.ds(h * STRIDE, n)]
        return buf.at[slot, pl.ds(0, n)], dst, ssem[slot]

    def fire_gather(t, slot):
        pltpu.async_copy(*_gather_args(t, slot))

    def wait_gather(t, slot):
        pltpu.make_async_copy(*_gather_args(t, slot)).wait()

    def fire_store(t, slot, h):
        pltpu.async_copy(*_store_args(t, slot, h))

    def wait_store(t, slot, h):
        pltpu.make_async_copy(*_store_args(t, slot, h)).wait()

    def add_pos(slot, h):
        def add_rows(g, carry):
            for u in range(2):
                r = g * 2 + u
                for cc in range(D_MODEL // LANES):
                    sl = pl.ds(cc * LANES, LANES)
                    plsc.addupdate(buf.at[slot, r, sl],
                                   pos_v[h * STRIDE + r, sl])
            return carry
        lax.fori_loop(0, (CLEN if h else STRIDE) // 2, add_rows, 0)

    for b in range(NSLOT - 1):
        fire_gather(b, b)

    def outer(g, carry):
        for b in range(NSLOT):
            h = b & 1
            t = g * NSLOT + b
            wait_gather(t, b)
            add_pos(b, h)
            fire_store(t, b, h)
            nxt = t + NSLOT - 1
            ns = (b + NSLOT - 1) % NSLOT

            @pl.when(nxt < CHUNKS)
            def _fire():
                @pl.when(t >= 1)
                def _drain():
                    wait_store(t - 1, ns, ns & 1)
                fire_gather(nxt, ns)
        return carry

    lax.fori_loop(0, CHUNKS // NSLOT, outer, 0)

    for b in range(NSLOT):
        wait_store(CHUNKS - NSLOT + b, b, b & 1)


@jax.jit
def kernel(input_ids, W, pos_table):
    ids = jnp.concatenate(
        [input_ids[:, :CLEN], input_ids[:, STRIDE:]], axis=1
    ).reshape(BATCH * 2, CLEN)
    pos = pos_table[:SEQ]
    run = pl.kernel(
        _emb_body,
        mesh=plsc.VectorSubcoreMesh(core_axis_name="c", subcore_axis_name="s"),
        out_type=jax.ShapeDtypeStruct((BATCH, SEQ, D_MODEL), jnp.float32),
        scratch_types=[
            pltpu.VMEM((CHUNKS, CLEN), jnp.int32),
            pltpu.VMEM((SEQ, D_MODEL), jnp.float32),
            pltpu.VMEM((NSLOT, CLEN, D_MODEL), jnp.float32),
        ] + [pltpu.SemaphoreType.DMA] * (2 * NSLOT),
    )
    return run(ids, W, pos)


# speedup vs baseline: 1.9201x; 1.0006x over previous
import jax
import jax.numpy as jnp
from jax import lax
from jax.experimental import pallas as pl
from jax.experimental.pallas import tpu as pltpu
from jax.experimental.pallas import tpu_sc as plsc

N_CORES = 2
N_SUBCORES = 16
N_WORKERS = N_CORES * N_SUBCORES

BATCH = 1024
SEQ = 200
D_MODEL = 128
SEQ_PER_W = BATCH // N_WORKERS
CHUNKS = 2 * SEQ_PER_W
CLEN = 104
STRIDE = 96
NSLOT = 4
LANES = 16


def _emb_body(ids_hbm, w_hbm, pos_hbm, out_hbm,
              idx_all, pos_v, buf,
              g0, g1, g2, g3, s0, s1, s2, s3):
    gsem = (g0, g1, g2, g3)
    ssem = (s0, s1, s2, s3)
    c = lax.axis_index("c")
    s = lax.axis_index("s")
    wid = s * N_CORES + c
    base = wid * CHUNKS

    pltpu.sync_copy(ids_hbm.at[pl.ds(base, CHUNKS)], idx_all)
    pltpu.sync_copy(pos_hbm, pos_v)

    def _gather_args(t, slot):
        return w_hbm.at[idx_all.at[t]], buf.at[slot], gsem[slot]

    def _store_args(t, slot, h):
        n = CLEN if h else STRIDE
        seq = wid * SEQ_PER_W + (t >> 1)
        dst = out_hbm.at[seq, pl.ds(h * STRIDE, n)]
        return buf.at[slot, pl.ds(0, n)], dst, ssem[slot]

    def fire_gather(t, slot):
        pltpu.async_copy(*_gather_args(t, slot))

    def wait_gather(t, slot):
        pltpu.make_async_copy(*_gather_args(t, slot)).wait()

    def fire_store(t, slot, h):
        pltpu.async_copy(*_store_args(t, slot, h))

    def wait_store(t, slot, h):
        pltpu.make_async_copy(*_store_args(t, slot, h)).wait()

    def add_pos(slot, h):
        def add_rows(g, carry):
            for u in range(4):
                r = g * 4 + u
                for cc in range(D_MODEL // LANES):
                    sl = pl.ds(cc * LANES, LANES)
                    plsc.addupdate(buf.at[slot, r, sl],
                                   pos_v[h * STRIDE + r, sl])
            return carry
        lax.fori_loop(0, (CLEN if h else STRIDE) // 4, add_rows, 0)

    for b in range(NSLOT - 1):
        fire_gather(b, b)

    def outer(g, carry):
        for b in range(NSLOT):
            h = b & 1
            t = g * NSLOT + b
            wait_gather(t, b)
            add_pos(b, h)
            fire_store(t, b, h)
            nxt = t + NSLOT - 1
            ns = (b + NSLOT - 1) % NSLOT

            @pl.when(nxt < CHUNKS)
            def _fire():
                @pl.when(t >= 1)
                def _drain():
                    wait_store(t - 1, ns, ns & 1)
                fire_gather(nxt, ns)
        return carry

    lax.fori_loop(0, CHUNKS // NSLOT, outer, 0)

    for b in range(NSLOT):
        wait_store(CHUNKS - NSLOT + b, b, b & 1)


@jax.jit
def kernel(input_ids, W, pos_table):
    ids = jnp.concatenate(
        [input_ids[:, :CLEN], input_ids[:, STRIDE:]], axis=1
    ).reshape(BATCH * 2, CLEN)
    pos = pos_table[:SEQ]
    run = pl.kernel(
        _emb_body,
        mesh=plsc.VectorSubcoreMesh(core_axis_name="c", subcore_axis_name="s"),
        out_type=jax.ShapeDtypeStruct((BATCH, SEQ, D_MODEL), jnp.float32),
        scratch_types=[
            pltpu.VMEM((CHUNKS, CLEN), jnp.int32),
            pltpu.VMEM((SEQ, D_MODEL), jnp.float32),
            pltpu.VMEM((NSLOT, CLEN, D_MODEL), jnp.float32),
        ] + [pltpu.SemaphoreType.DMA] * (2 * NSLOT),
    )
    return run(ids, W, pos)
